# baseline (speedup 1.0000x reference)
import functools

import jax
import jax.numpy as jnp
from jax.experimental import pallas as pl
from jax.experimental.pallas import tpu as pltpu

_K = 16
_NSLICE = 16


def _threshold(s, depth):
    b, n = s.shape
    w = n // _NSLICE
    neg = jnp.float32(-jnp.inf)
    g = [s[:, 0:w]] + [jnp.full((b, w), neg, jnp.float32)] * (depth - 1)
    for i in range(1, _NSLICE):
        t = s[:, i * w:(i + 1) * w]
        for lvl in range(depth - 1):
            nxt = jnp.minimum(g[lvl], t)
            g[lvl] = jnp.maximum(g[lvl], t)
            t = nxt
        g[depth - 1] = jnp.maximum(g[depth - 1], t)
    e = jnp.concatenate(g, axis=1)
    t_lb = jnp.float32(jnp.inf)
    for _ in range(_K):
        t_lb = jnp.max(e, axis=1, keepdims=True)
        e = jnp.where(e == t_lb, neg, e)
    return t_lb


def _sim_block(xb_ref, xf_ref):
    return jax.lax.dot_general(
        xb_ref[...],
        xf_ref[...],
        (((1,), (1,)), ((), ())),
        preferred_element_type=jnp.float32,
    )


_DEPTH = 4


def _fast_kernel(xb_ref, xf_ref, o_ref, fl_ref):
    xb = xb_ref[...]
    b = xb.shape[0]
    n = o_ref.shape[1]
    w = n // _NSLICE
    neg = jnp.float32(-jnp.inf)
    sl = []
    g = [jnp.full((b, w), neg, jnp.float32) for _ in range(_DEPTH)]
    for i in range(_NSLICE):
        t = jax.lax.dot_general(
            xb,
            xf_ref[i * w:(i + 1) * w, :],
            (((1,), (1,)), ((), ())),
            preferred_element_type=jnp.float32,
        )
        sl.append(t)
        for lvl in range(_DEPTH - 1):
            nxt = jnp.minimum(g[lvl], t)
            g[lvl] = jnp.maximum(g[lvl], t)
            t = nxt
        g[_DEPTH - 1] = jnp.maximum(g[_DEPTH - 1], t)
    e = jnp.concatenate(g, axis=1)
    t_cut = jnp.float32(jnp.inf)
    for _ in range(_K):
        t_cut = jnp.max(e, axis=1, keepdims=True)
        e = jnp.where(e == t_cut, neg, e)
    c = jnp.zeros((b, 1), dtype=jnp.int32)
    for i in range(_NSLICE):
        cand = sl[i] >= t_cut
        o_ref[:, i * w:(i + 1) * w] = jnp.where(cand, sl[i], jnp.float32(0.0))
        c = c + jnp.sum(cand.astype(jnp.int32), axis=1, keepdims=True)
    bad = jnp.max(jnp.abs(c - _K))
    fl_ref[...] = jnp.broadcast_to(bad, fl_ref.shape)


def _exact_kernel(xb_ref, xf_ref, o_ref, mk_ref):
    s = _sim_block(xb_ref, xf_ref)
    b, n = s.shape
    pos = jnp.float32(jnp.inf)
    t_lb = _threshold(s, 2)
    cand = s >= t_lb
    mk = cand.astype(jnp.int32)
    mk_ref[...] = mk
    o_ref[...] = jnp.where(cand, s, jnp.float32(0.0))
    c0 = jnp.sum(mk, axis=1, keepdims=True)

    iota = jax.lax.broadcasted_iota(jnp.int32, (b, n), 1)

    def _body(_):
        mk = mk_ref[...]
        c = jnp.sum(mk, axis=1, keepdims=True)
        needs = c > _K
        candb = mk > 0
        mv = jnp.min(jnp.where(candb, s, pos), axis=1, keepdims=True)
        tied = candb & (s == mv)
        last = jnp.max(jnp.where(tied, iota, -1), axis=1, keepdims=True)
        remove = needs & (iota == last)
        mk_ref[...] = jnp.where(remove, 0, mk)
        o_ref[...] = jnp.where(remove, jnp.float32(0.0), o_ref[...])
        c = c - needs.astype(jnp.int32)
        return jnp.max(c) > _K

    jax.lax.while_loop(lambda p: p, _body, jnp.max(c0) > _K)


def _common_specs(n, d, block_rows):
    return dict(
        grid=(n // block_rows,),
        in_specs=[
            pl.BlockSpec((block_rows, d), lambda i: (i, 0)),
            pl.BlockSpec((n, d), lambda i: (0, 0)),
        ],
        compiler_params=pltpu.CompilerParams(
            dimension_semantics=("parallel",)
        ),
    )


def _knn_fast(x, block_rows, interpret=False):
    n, d = x.shape
    nb = n // block_rows
    return pl.pallas_call(
        _fast_kernel,
        out_specs=[
            pl.BlockSpec((block_rows, n), lambda i: (i, 0)),
            pl.BlockSpec((8, 128), lambda i: (i, 0)),
        ],
        out_shape=[
            jax.ShapeDtypeStruct((n, n), jnp.float32),
            jax.ShapeDtypeStruct((nb * 8, 128), jnp.int32),
        ],
        interpret=interpret,
        **_common_specs(n, d, block_rows),
    )(x, x)


def _knn_exact(x, block_rows, interpret=False):
    n, d = x.shape
    return pl.pallas_call(
        _exact_kernel,
        out_specs=pl.BlockSpec((block_rows, n), lambda i: (i, 0)),
        out_shape=jax.ShapeDtypeStruct((n, n), jnp.float32),
        scratch_shapes=[pltpu.VMEM((block_rows, n), jnp.int32)],
        interpret=interpret,
        **_common_specs(n, d, block_rows),
    )(x, x)


@functools.partial(jax.jit, static_argnames=("block_rows", "interpret"))
def _knn_adj(x, block_rows=512, interpret=False):
    o, fl = _knn_fast(x, block_rows, interpret)
    return jax.lax.cond(
        jnp.max(fl) > 0,
        lambda xx: _knn_exact(xx, block_rows, interpret),
        lambda xx: o,
        x,
    )


def kernel(x):
    return (x, _knn_adj(x))


# speedup vs baseline: 1.0684x; 1.0684x over previous
import functools

import jax
import jax.numpy as jnp
from jax.experimental import pallas as pl
from jax.experimental.pallas import tpu as pltpu

_K = 16
_NSLICE = 16


def _threshold(s, depth):
    b, n = s.shape
    w = n // _NSLICE
    neg = jnp.float32(-jnp.inf)
    g = [s[:, 0:w]] + [jnp.full((b, w), neg, jnp.float32)] * (depth - 1)
    for i in range(1, _NSLICE):
        t = s[:, i * w:(i + 1) * w]
        for lvl in range(depth - 1):
            nxt = jnp.minimum(g[lvl], t)
            g[lvl] = jnp.maximum(g[lvl], t)
            t = nxt
        g[depth - 1] = jnp.maximum(g[depth - 1], t)
    t_lb = jnp.float32(jnp.inf)
    for _ in range(_K):
        h = g[0]
        for lvl in range(1, depth):
            h = jnp.maximum(h, g[lvl])
        t_lb = jnp.max(h, axis=1, keepdims=True)
        g = [jnp.where(x == t_lb, neg, x) for x in g]
    return t_lb


def _sim_block(xb_ref, xf_ref):
    return jax.lax.dot_general(
        xb_ref[...],
        xf_ref[...],
        (((1,), (1,)), ((), ())),
        preferred_element_type=jnp.float32,
    )


_DEPTH = 4


def _fast_kernel(xb_ref, xf_ref, o_ref, fl_ref):
    s = _sim_block(xb_ref, xf_ref)
    t = _threshold(s, _DEPTH)
    cand = s >= t
    o_ref[...] = jnp.where(cand, s, jnp.float32(0.0))
    c = jnp.sum(cand.astype(jnp.int32), axis=1, keepdims=True)
    bad = jnp.max(jnp.abs(c - _K))
    fl_ref[...] = jnp.broadcast_to(bad, fl_ref.shape)


def _exact_kernel(xb_ref, xf_ref, o_ref, mk_ref):
    s = _sim_block(xb_ref, xf_ref)
    b, n = s.shape
    pos = jnp.float32(jnp.inf)
    t_lb = _threshold(s, 2)
    cand = s >= t_lb
    mk = cand.astype(jnp.int32)
    mk_ref[...] = mk
    o_ref[...] = jnp.where(cand, s, jnp.float32(0.0))
    c0 = jnp.sum(mk, axis=1, keepdims=True)

    iota = jax.lax.broadcasted_iota(jnp.int32, (b, n), 1)

    def _body(_):
        mk = mk_ref[...]
        c = jnp.sum(mk, axis=1, keepdims=True)
        needs = c > _K
        candb = mk > 0
        mv = jnp.min(jnp.where(candb, s, pos), axis=1, keepdims=True)
        tied = candb & (s == mv)
        last = jnp.max(jnp.where(tied, iota, -1), axis=1, keepdims=True)
        remove = needs & (iota == last)
        mk_ref[...] = jnp.where(remove, 0, mk)
        o_ref[...] = jnp.where(remove, jnp.float32(0.0), o_ref[...])
        c = c - needs.astype(jnp.int32)
        return jnp.max(c) > _K

    jax.lax.while_loop(lambda p: p, _body, jnp.max(c0) > _K)


def _common_specs(n, d, block_rows):
    return dict(
        grid=(n // block_rows,),
        in_specs=[
            pl.BlockSpec((block_rows, d), lambda i: (i, 0)),
            pl.BlockSpec((n, d), lambda i: (0, 0)),
        ],
        compiler_params=pltpu.CompilerParams(
            dimension_semantics=("parallel",)
        ),
    )


def _knn_fast(x, block_rows, interpret=False):
    n, d = x.shape
    nb = n // block_rows
    return pl.pallas_call(
        _fast_kernel,
        out_specs=[
            pl.BlockSpec((block_rows, n), lambda i: (i, 0)),
            pl.BlockSpec((8, 128), lambda i: (i, 0)),
        ],
        out_shape=[
            jax.ShapeDtypeStruct((n, n), jnp.float32),
            jax.ShapeDtypeStruct((nb * 8, 128), jnp.int32),
        ],
        interpret=interpret,
        **_common_specs(n, d, block_rows),
    )(x, x)


def _knn_exact(x, block_rows, interpret=False):
    n, d = x.shape
    return pl.pallas_call(
        _exact_kernel,
        out_specs=pl.BlockSpec((block_rows, n), lambda i: (i, 0)),
        out_shape=jax.ShapeDtypeStruct((n, n), jnp.float32),
        scratch_shapes=[pltpu.VMEM((block_rows, n), jnp.int32)],
        interpret=interpret,
        **_common_specs(n, d, block_rows),
    )(x, x)


@functools.partial(jax.jit, static_argnames=("block_rows", "interpret"))
def _knn_adj(x, block_rows=512, interpret=False):
    o, fl = _knn_fast(x, block_rows, interpret)
    return jax.lax.cond(
        jnp.max(fl) > 0,
        lambda xx: _knn_exact(xx, block_rows, interpret),
        lambda xx: o,
        x,
    )


def kernel(x):
    return (x, _knn_adj(x))
